# Initial kernel scaffold; baseline (speedup 1.0000x reference)
import functools

import jax
import jax.numpy as jnp
from jax import lax
from jax.experimental import pallas as pl
from jax.experimental.pallas import tpu as pltpu
from jax.experimental.pallas import tpu_sc as plsc

N = 10000
C = 128
E = 320000
NUM_ITERS = 10
ALPHA = 0.1

NC = 2
NS = 16
NW = NC * NS

CHUNK = 128
EDGES_PER_TILE = E // NW
NCHUNK = -(-EDGES_PER_TILE // CHUNK)
NCHUNK += NCHUNK % 2
CAP = NCHUNK * CHUNK
N_PAD = ((N + NS * CHUNK - 1) // (NS * CHUNK)) * (NS * CHUNK)
ROWS_PER_TILE = N_PAD // NS
DUMP = N


def _sc_scatter_body(src_slab, dst_slab, est, zeros_rows, out,
                     src_v, dst_v, buf_a, buf_b, acc, sem_a, sem_b):
    c = lax.axis_index("c")
    s = lax.axis_index("s")
    wid = c * NS + s

    pltpu.sync_copy(src_slab.at[wid], src_v)
    pltpu.sync_copy(dst_slab.at[wid], dst_v)

    base = s * ROWS_PER_TILE
    for k in range(ROWS_PER_TILE // CHUNK):
        pltpu.sync_copy(zeros_rows, acc.at[pl.ds(base + k * CHUNK, CHUNK)])
    plsc.subcore_barrier()

    def body(i, _):
        j0 = 2 * i
        j1 = j0 + 1
        cp0 = pltpu.async_copy(est.at[src_v.at[j0]], buf_a, sem_a)
        cp1 = pltpu.async_copy(est.at[src_v.at[j1]], buf_b, sem_b)
        cp0.wait()
        pltpu.sync_copy(buf_a, acc.at[dst_v.at[j0]], add=True)
        cp1.wait()
        pltpu.sync_copy(buf_b, acc.at[dst_v.at[j1]], add=True)
        return 0

    lax.fori_loop(0, NCHUNK // 2, body, 0)
    plsc.subcore_barrier()

    pltpu.sync_copy(acc.at[pl.ds(base, ROWS_PER_TILE)],
                    out.at[c, pl.ds(base, ROWS_PER_TILE)])


_sc_scatter = functools.partial(
    pl.kernel,
    _sc_scatter_body,
    out_type=jax.ShapeDtypeStruct((NC, N_PAD, C), jnp.float32),
    mesh=plsc.VectorSubcoreMesh(core_axis_name="c", subcore_axis_name="s"),
    scratch_types=[
        pltpu.VMEM((NCHUNK, CHUNK), jnp.int32),
        pltpu.VMEM((NCHUNK, CHUNK), jnp.int32),
        pltpu.VMEM((CHUNK, C), jnp.float32),
        pltpu.VMEM((CHUNK, C), jnp.float32),
        pltpu.VMEM_SHARED((N_PAD, C), jnp.float32),
        pltpu.SemaphoreType.DMA,
        pltpu.SemaphoreType.DMA,
    ],
)()


ROW_BLK = 1000


def _tc_fuse_body(s_ref, d_ref, w_ref, init_ref, out_ref):
    w = w_ref[...]
    w_max = jnp.max(w, axis=1, keepdims=True)
    e = jnp.exp(w - w_max)
    p = e / jnp.sum(e, axis=1, keepdims=True)

    sagg = s_ref[0] + s_ref[1]
    deg = d_ref[0] + d_ref[1]
    norm = 1.0 / jnp.maximum(deg, 1.0)
    prop = jnp.dot(sagg, p, preferred_element_type=jnp.float32)
    out_ref[...] = (1.0 - ALPHA) * norm * prop + ALPHA * init_ref[...]


_tc_fuse = pl.pallas_call(
    _tc_fuse_body,
    grid=(N // ROW_BLK,),
    in_specs=[
        pl.BlockSpec((NC, ROW_BLK, C), lambda i: (0, i, 0)),
        pl.BlockSpec((NC, ROW_BLK, C), lambda i: (0, i, 0)),
        pl.BlockSpec((C, C), lambda i: (0, 0)),
        pl.BlockSpec((ROW_BLK, C), lambda i: (i, 0)),
    ],
    out_specs=pl.BlockSpec((ROW_BLK, C), lambda i: (i, 0)),
    out_shape=jax.ShapeDtypeStruct((N, C), jnp.float32),
)


def kernel(edge_index, estimates, W):
    src = edge_index[0].astype(jnp.int32)
    dst = edge_index[1].astype(jnp.int32)

    pad = CAP - EDGES_PER_TILE
    src_slab = jnp.concatenate(
        [src.reshape(NW, EDGES_PER_TILE),
         jnp.zeros((NW, pad), jnp.int32)], axis=1).reshape(NW, NCHUNK, CHUNK)
    dst_slab = jnp.concatenate(
        [dst.reshape(NW, EDGES_PER_TILE),
         jnp.full((NW, pad), DUMP, jnp.int32)], axis=1).reshape(NW, NCHUNK, CHUNK)

    zeros_rows = jnp.zeros((CHUNK, C), jnp.float32)
    ones_tbl = jnp.ones((N, C), jnp.float32)

    deg_p = _sc_scatter(src_slab, dst_slab, ones_tbl, zeros_rows)

    est = estimates
    for _ in range(NUM_ITERS):
        s_p = _sc_scatter(src_slab, dst_slab, est, zeros_rows)
        est = _tc_fuse(s_p, deg_p, W, estimates)
    return est


# speedup vs baseline: 4.5617x; 4.5617x over previous
import jax
import jax.numpy as jnp
from jax import lax
from jax.experimental import pallas as pl
from jax.experimental.pallas import tpu as pltpu
from jax.experimental.pallas import tpu_sc as plsc

N = 10000
C = 128
E = 320000
NUM_ITERS = 10
ALPHA = 0.1

NC = 2
NS = 16
NW = NC * NS

CHUNK = 128
EDGES_PER_TILE = E // NW
NCHUNK = -(-EDGES_PER_TILE // CHUNK)
CAP = NCHUNK * CHUNK
N_PAD = ((N + NS * 128 - 1) // (NS * 128)) * (NS * 128)
ROWS_PER_TILE = N_PAD // NS
DUMP = N


def _sc_scatter_body(src_slab, dst_slab, est, out, acc, sem_a):
    def inner(src_v, dst_v, buf_a):
        _sc_scatter_inner(src_slab, dst_slab, est, out, acc,
                          sem_a, src_v, dst_v, buf_a)

    pl.run_scoped(
        inner,
        pltpu.VMEM((NCHUNK, CHUNK), jnp.int32),
        pltpu.VMEM((NCHUNK, CHUNK), jnp.int32),
        pltpu.VMEM((CHUNK, C), jnp.float32),
    )


def _sc_scatter_inner(src_slab, dst_slab, est, out, acc,
                      sem_a, src_v, dst_v, buf_a):
    c = lax.axis_index("c")
    s = lax.axis_index("s")
    wid = c * NS + s

    pltpu.sync_copy(src_slab.at[wid], src_v)
    pltpu.sync_copy(dst_slab.at[wid], dst_v)

    zero16 = jnp.zeros((16,), jnp.float32)

    def zrow(r, _):
        for k in range(C // 16):
            buf_a[r, pl.ds(k * 16, 16)] = zero16
        return 0

    lax.fori_loop(0, CHUNK, zrow, 0)
    base = s * ROWS_PER_TILE

    def zcopy(k, _):
        pltpu.sync_copy(buf_a, acc.at[pl.ds(base + k * CHUNK, CHUNK)])
        return 0

    lax.fori_loop(0, ROWS_PER_TILE // CHUNK, zcopy, 0)
    plsc.subcore_barrier()

    def body(j, _):
        cp = pltpu.async_copy(est.at[src_v.at[j]], buf_a, sem_a)
        cp.wait()
        pltpu.sync_copy(buf_a, acc.at[dst_v.at[j]], add=True)
        return 0

    lax.fori_loop(0, NCHUNK, body, 0)
    plsc.subcore_barrier()

    def fcopy(k, _):
        pltpu.sync_copy(acc.at[pl.ds(base + k * CHUNK, CHUNK)],
                        out.at[c, pl.ds(base + k * CHUNK, CHUNK)])
        return 0

    lax.fori_loop(0, ROWS_PER_TILE // CHUNK, fcopy, 0)


_sc_scatter = pl.kernel(
    _sc_scatter_body,
    out_type=jax.ShapeDtypeStruct((NC, N_PAD, C), jnp.float32),
    mesh=plsc.VectorSubcoreMesh(core_axis_name="c", subcore_axis_name="s"),
    scratch_types=[
        pltpu.VMEM_SHARED((N_PAD, C), jnp.float32),
        pltpu.SemaphoreType.DMA,
    ],
    compiler_params=pltpu.CompilerParams(use_tc_tiling_on_sc=True),
)


ROW_BLK = 1000


def _tc_fuse_body(s_ref, d_ref, w_ref, init_ref, out_ref):
    w = w_ref[...]
    w_max = jnp.max(w, axis=1, keepdims=True)
    e = jnp.exp(w - w_max)
    p = e / jnp.sum(e, axis=1, keepdims=True)

    sagg = s_ref[0] + s_ref[1]
    deg = d_ref[0] + d_ref[1]
    norm = 1.0 / jnp.maximum(deg, 1.0)
    prop = jnp.dot(sagg, p, preferred_element_type=jnp.float32)
    out_ref[...] = (1.0 - ALPHA) * norm * prop + ALPHA * init_ref[...]


_tc_fuse = pl.pallas_call(
    _tc_fuse_body,
    grid=(N // ROW_BLK,),
    in_specs=[
        pl.BlockSpec((NC, ROW_BLK, C), lambda i: (0, i, 0)),
        pl.BlockSpec((NC, ROW_BLK, C), lambda i: (0, i, 0)),
        pl.BlockSpec((C, C), lambda i: (0, 0)),
        pl.BlockSpec((ROW_BLK, C), lambda i: (i, 0)),
    ],
    out_specs=pl.BlockSpec((ROW_BLK, C), lambda i: (i, 0)),
    out_shape=jax.ShapeDtypeStruct((N, C), jnp.float32),
)


def kernel(edge_index, estimates, W):
    src = edge_index[0].astype(jnp.int32)
    dst = edge_index[1].astype(jnp.int32)

    pad = CAP - EDGES_PER_TILE
    src_slab = jnp.concatenate(
        [src.reshape(NW, EDGES_PER_TILE),
         jnp.zeros((NW, pad), jnp.int32)], axis=1).reshape(NW, NCHUNK, CHUNK)
    dst_slab = jnp.concatenate(
        [dst.reshape(NW, EDGES_PER_TILE),
         jnp.full((NW, pad), DUMP, jnp.int32)], axis=1).reshape(NW, NCHUNK, CHUNK)

    ones_tbl = jnp.ones((N, C), jnp.float32)

    deg_p = _sc_scatter(src_slab, dst_slab, ones_tbl)

    est = estimates
    for _ in range(NUM_ITERS):
        s_p = _sc_scatter(src_slab, dst_slab, est)
        est = _tc_fuse(s_p, deg_p, W, estimates)
    return est
